# baseline (speedup 1.0000x reference)
import functools

import jax
import jax.numpy as jnp
from jax import lax
from jax.experimental import pallas as pl
from jax.experimental.pallas import tpu as pltpu
from jax.experimental.pallas import tpu_sc as plsc

_F = 16
_IDX_CHUNK = 128


_NP = 1 << 20
_BLK = 1 << 18


def _flatten(tab_t):
    F, N = tab_t.shape
    n_col = _NP // _BLK

    def body(t_ref, o_ref):
        k = pl.program_id(2)
        o_ref[...] = t_ref[pl.ds(k, 1), :].reshape(_BLK)

    return pl.pallas_call(
        body,
        grid=(F // 8, n_col, 8),
        in_specs=[pl.BlockSpec((8, _BLK), lambda rb, c, k: (rb, c))],
        out_specs=pl.BlockSpec((_BLK,),
                               lambda rb, c, k: ((rb * 8 + k) * n_col + c)),
        out_shape=jax.ShapeDtypeStruct((F * _NP,), jnp.float32),
    )(tab_t)


@functools.cache
def _make_gather(B, N):
    info = plsc.get_sparse_core_info()
    nc, ns = info.num_cores, info.num_subcores
    nw = nc * ns
    b_per_w = B // nw
    n_chunks = b_per_w // _IDX_CHUNK
    mesh = plsc.VectorSubcoreMesh(core_axis_name="c", subcore_axis_name="s")

    @functools.partial(
        pl.kernel,
        mesh=mesh,
        compiler_params=pltpu.CompilerParams(use_tc_tiling_on_sc=False),
        out_type=jax.ShapeDtypeStruct((_F, B), jnp.float32),
        scratch_types=[
            pltpu.VMEM((n_chunks, _IDX_CHUNK), jnp.int32),
            pltpu.VMEM((_F, b_per_w), jnp.float32),
            pltpu.SemaphoreType.DMA,
        ],
    )
    def gather_k(idx_hbm, tab_hbm, out, idx_v, rows_v, sem):
        wid = lax.axis_index("s") * nc + lax.axis_index("c")
        base = wid * b_per_w
        for j in range(n_chunks):
            pltpu.sync_copy(idx_hbm.at[pl.ds(base + j * _IDX_CHUNK,
                                             _IDX_CHUNK)], idx_v.at[j])
        copies = [
            pltpu.async_copy(
                tab_hbm.at[f].at[idx_v.at[j]],
                rows_v.at[f, pl.ds(j * _IDX_CHUNK, _IDX_CHUNK)], sem)
            for j in range(n_chunks)
            for f in range(_F)
        ]
        for c in copies:
            c.wait()
        pltpu.sync_copy(rows_v, out.at[:, pl.ds(base, b_per_w)])

    return gather_k


def _mlp_pallas(gu, gi, W1u, W1i, b1, W2, b2, W3, b3):
    F, B = gu.shape
    blk = 2048
    n1 = W1u.shape[1]
    n2 = W2.shape[1]
    dn = (((0,), (0,)), ((), ()))

    def body(gu_ref, gi_ref, w1u_ref, w1i_ref, b1_ref,
             w2_ref, b2_ref, w3_ref, b3_ref, out_ref):
        h = (lax.dot_general(gu_ref[...], w1u_ref[...], dn,
                             preferred_element_type=jnp.float32)
             + lax.dot_general(gi_ref[...], w1i_ref[...], dn,
                               preferred_element_type=jnp.float32)
             + b1_ref[...])
        h = jnp.dot(h, w2_ref[...], preferred_element_type=jnp.float32) \
            + b2_ref[...]
        o = jnp.dot(h, w3_ref[...], preferred_element_type=jnp.float32) \
            + b3_ref[...]
        out_ref[...] = 1.0 / (1.0 + jnp.exp(-o))

    return pl.pallas_call(
        body,
        grid=(B // blk,),
        in_specs=[
            pl.BlockSpec((F, blk), lambda i: (0, i)),
            pl.BlockSpec((F, blk), lambda i: (0, i)),
            pl.BlockSpec((F, n1), lambda i: (0, 0)),
            pl.BlockSpec((F, n1), lambda i: (0, 0)),
            pl.BlockSpec((1, n1), lambda i: (0, 0)),
            pl.BlockSpec((n1, n2), lambda i: (0, 0)),
            pl.BlockSpec((1, n2), lambda i: (0, 0)),
            pl.BlockSpec((n2, 1), lambda i: (0, 0)),
            pl.BlockSpec((1, 1), lambda i: (0, 0)),
        ],
        out_specs=pl.BlockSpec((blk, 1), lambda i: (i, 0)),
        out_shape=jax.ShapeDtypeStruct((B, 1), jnp.float32),
    )(gu, gi, W1u, W1i, b1.reshape(1, n1), W2, b2.reshape(1, n2),
      W3, b3.reshape(1, 1))


def kernel(user, item, user_emb, item_emb, W1, b1, W2, b2, W3, b3):
    B = user.shape[0]
    N = user_emb.shape[0]
    gather = _make_gather(B, _NP)
    u_flat = _flatten(user_emb.T)
    gu = gather(user.astype(jnp.int32), u_flat.reshape(_F, _NP))
    i_flat = _flatten(item_emb.T)
    gi = gather(item.astype(jnp.int32), i_flat.reshape(_F, _NP))
    return _mlp_pallas(gu, gi, W1[:_F], W1[_F:], b1, W2, b2, W3, b3)


# speedup vs baseline: 1.1019x; 1.1019x over previous
import functools

import jax
import jax.numpy as jnp
from jax import lax
from jax.experimental import pallas as pl
from jax.experimental.pallas import tpu as pltpu
from jax.experimental.pallas import tpu_sc as plsc

_F = 16
_IDX_CHUNK = 128


_NP = 1 << 20
_BLK = 1 << 18


def _flatten(ut, it):
    F, N = ut.shape
    n_col = _NP // _BLK

    def body(u_ref, i_ref, uo_ref, io_ref):
        k = pl.program_id(2)
        uo_ref[...] = u_ref[pl.ds(k, 1), :].reshape(_BLK)
        io_ref[...] = i_ref[pl.ds(k, 1), :].reshape(_BLK)

    in_spec = pl.BlockSpec((8, _BLK), lambda rb, c, k: (rb, c))
    out_spec = pl.BlockSpec((_BLK,),
                            lambda rb, c, k: ((rb * 8 + k) * n_col + c))
    return pl.pallas_call(
        body,
        grid=(F // 8, n_col, 8),
        in_specs=[in_spec, in_spec],
        out_specs=(out_spec, out_spec),
        out_shape=(jax.ShapeDtypeStruct((F * _NP,), jnp.float32),
                   jax.ShapeDtypeStruct((F * _NP,), jnp.float32)),
    )(ut, it)


@functools.cache
def _make_gather(B, N):
    info = plsc.get_sparse_core_info()
    nc, ns = info.num_cores, info.num_subcores
    nw = nc * ns
    b_per_w = B // nw
    n_chunks = b_per_w // _IDX_CHUNK
    mesh = plsc.VectorSubcoreMesh(core_axis_name="c", subcore_axis_name="s")

    @functools.partial(
        pl.kernel,
        mesh=mesh,
        compiler_params=pltpu.CompilerParams(use_tc_tiling_on_sc=False),
        out_type=(
            jax.ShapeDtypeStruct((_F, B), jnp.float32),
            jax.ShapeDtypeStruct((_F, B), jnp.float32),
        ),
        scratch_types=[
            pltpu.VMEM((n_chunks, _IDX_CHUNK), jnp.int32),
            pltpu.VMEM((n_chunks, _IDX_CHUNK), jnp.int32),
            pltpu.VMEM((_F, b_per_w), jnp.float32),
            pltpu.VMEM((_F, b_per_w), jnp.float32),
            pltpu.SemaphoreType.DMA,
        ],
    )
    def gather_k(user_hbm, item_hbm, ut_hbm, it_hbm, u_out, i_out,
                 uidx_v, iidx_v, urows_v, irows_v, sem):
        wid = lax.axis_index("s") * nc + lax.axis_index("c")
        base = wid * b_per_w
        for j in range(n_chunks):
            pltpu.sync_copy(user_hbm.at[pl.ds(base + j * _IDX_CHUNK,
                                              _IDX_CHUNK)], uidx_v.at[j])
            pltpu.sync_copy(item_hbm.at[pl.ds(base + j * _IDX_CHUNK,
                                              _IDX_CHUNK)], iidx_v.at[j])
        copies = [
            pltpu.async_copy(
                tab.at[f].at[idx.at[j]],
                rows.at[f, pl.ds(j * _IDX_CHUNK, _IDX_CHUNK)], sem)
            for j in range(n_chunks)
            for f in range(_F)
            for tab, idx, rows in ((ut_hbm, uidx_v, urows_v),
                                   (it_hbm, iidx_v, irows_v))
        ]
        for c in copies:
            c.wait()
        pltpu.sync_copy(urows_v, u_out.at[:, pl.ds(base, b_per_w)])
        pltpu.sync_copy(irows_v, i_out.at[:, pl.ds(base, b_per_w)])

    return gather_k


def _mlp_pallas(gu, gi, W1u, W1i, b1, W2, b2, W3, b3):
    F, B = gu.shape
    blk = 2048
    n1 = W1u.shape[1]
    n2 = W2.shape[1]
    dn = (((0,), (0,)), ((), ()))

    def body(gu_ref, gi_ref, w1u_ref, w1i_ref, b1_ref,
             w2_ref, b2_ref, w3_ref, b3_ref, out_ref):
        h = (lax.dot_general(gu_ref[...], w1u_ref[...], dn,
                             preferred_element_type=jnp.float32)
             + lax.dot_general(gi_ref[...], w1i_ref[...], dn,
                               preferred_element_type=jnp.float32)
             + b1_ref[...])
        h = jnp.dot(h, w2_ref[...], preferred_element_type=jnp.float32) \
            + b2_ref[...]
        o = jnp.dot(h, w3_ref[...], preferred_element_type=jnp.float32) \
            + b3_ref[...]
        out_ref[...] = 1.0 / (1.0 + jnp.exp(-o))

    return pl.pallas_call(
        body,
        grid=(B // blk,),
        in_specs=[
            pl.BlockSpec((F, blk), lambda i: (0, i)),
            pl.BlockSpec((F, blk), lambda i: (0, i)),
            pl.BlockSpec((F, n1), lambda i: (0, 0)),
            pl.BlockSpec((F, n1), lambda i: (0, 0)),
            pl.BlockSpec((1, n1), lambda i: (0, 0)),
            pl.BlockSpec((n1, n2), lambda i: (0, 0)),
            pl.BlockSpec((1, n2), lambda i: (0, 0)),
            pl.BlockSpec((n2, 1), lambda i: (0, 0)),
            pl.BlockSpec((1, 1), lambda i: (0, 0)),
        ],
        out_specs=pl.BlockSpec((blk, 1), lambda i: (i, 0)),
        out_shape=jax.ShapeDtypeStruct((B, 1), jnp.float32),
    )(gu, gi, W1u, W1i, b1.reshape(1, n1), W2, b2.reshape(1, n2),
      W3, b3.reshape(1, 1))


def kernel(user, item, user_emb, item_emb, W1, b1, W2, b2, W3, b3):
    B = user.shape[0]
    N = user_emb.shape[0]
    u_flat, i_flat = _flatten(user_emb.T, item_emb.T)
    gather = _make_gather(B, _NP)
    gu, gi = gather(user.astype(jnp.int32), item.astype(jnp.int32),
                    u_flat.reshape(_F, _NP), i_flat.reshape(_F, _NP))
    return _mlp_pallas(gu, gi, W1[:_F], W1[_F:], b1, W2, b2, W3, b3)


# speedup vs baseline: 1.1092x; 1.0066x over previous
import functools

import jax
import jax.numpy as jnp
from jax import lax
from jax.experimental import pallas as pl
from jax.experimental.pallas import tpu as pltpu
from jax.experimental.pallas import tpu_sc as plsc

_F = 16
_IDX_CHUNK = 128


_NP = 1 << 20
_BLK = 1 << 18


def _flatten(ut, it):
    F, N = ut.shape
    n_col = _NP // _BLK

    def body(u_ref, i_ref, uo_ref, io_ref):
        k = pl.program_id(2)
        uo_ref[...] = u_ref[pl.ds(k, 1), :].reshape(_BLK)
        io_ref[...] = i_ref[pl.ds(k, 1), :].reshape(_BLK)

    in_spec = pl.BlockSpec((8, _BLK), lambda rb, c, k: (rb, c))
    out_spec = pl.BlockSpec((_BLK,),
                            lambda rb, c, k: ((rb * 8 + k) * n_col + c))
    return pl.pallas_call(
        body,
        grid=(F // 8, n_col, 8),
        in_specs=[in_spec, in_spec],
        out_specs=(out_spec, out_spec),
        out_shape=(jax.ShapeDtypeStruct((F * _NP,), jnp.float32),
                   jax.ShapeDtypeStruct((F * _NP,), jnp.float32)),
    )(ut, it)


@functools.cache
def _make_gather(B, N):
    info = plsc.get_sparse_core_info()
    nc, ns = info.num_cores, info.num_subcores
    nw = nc * ns
    b_per_w = B // nw
    n_chunks = b_per_w // _IDX_CHUNK
    mesh = plsc.VectorSubcoreMesh(core_axis_name="c", subcore_axis_name="s")

    @functools.partial(
        pl.kernel,
        mesh=mesh,
        compiler_params=pltpu.CompilerParams(use_tc_tiling_on_sc=False),
        out_type=(
            jax.ShapeDtypeStruct((_F, B), jnp.float32),
            jax.ShapeDtypeStruct((_F, B), jnp.float32),
        ),
        scratch_types=[
            pltpu.VMEM((n_chunks, _IDX_CHUNK), jnp.int32),
            pltpu.VMEM((n_chunks, _IDX_CHUNK), jnp.int32),
            pltpu.VMEM((_F, b_per_w), jnp.float32),
            pltpu.VMEM((_F, b_per_w), jnp.float32),
            pltpu.SemaphoreType.DMA,
        ],
    )
    def gather_k(user_hbm, item_hbm, ut_hbm, it_hbm, u_out, i_out,
                 uidx_v, iidx_v, urows_v, irows_v, sem):
        wid = lax.axis_index("s") * nc + lax.axis_index("c")
        base = wid * b_per_w
        for j in range(n_chunks):
            pltpu.sync_copy(user_hbm.at[pl.ds(base + j * _IDX_CHUNK,
                                              _IDX_CHUNK)], uidx_v.at[j])
            pltpu.sync_copy(item_hbm.at[pl.ds(base + j * _IDX_CHUNK,
                                              _IDX_CHUNK)], iidx_v.at[j])
        copies = [
            pltpu.async_copy(
                tab.at[f].at[idx.at[j]],
                rows.at[f, pl.ds(j * _IDX_CHUNK, _IDX_CHUNK)], sem)
            for j in range(n_chunks)
            for f in range(_F)
            for tab, idx, rows in ((ut_hbm, uidx_v, urows_v),
                                   (it_hbm, iidx_v, irows_v))
        ]
        for c in copies:
            c.wait()
        pltpu.sync_copy(urows_v, u_out.at[:, pl.ds(base, b_per_w)])
        pltpu.sync_copy(irows_v, i_out.at[:, pl.ds(base, b_per_w)])

    return gather_k


def _mlp_pallas(gu, gi, W1u, W1i, b1, W2, b2, W3, b3):
    F, B = gu.shape
    blk = 4096
    n1 = W1u.shape[1]
    n2 = W2.shape[1]
    dn = (((0,), (0,)), ((), ()))

    def body(gu_ref, gi_ref, w1u_ref, w1i_ref, b1_ref,
             w2_ref, b2_ref, w3_ref, b3_ref, out_ref):
        h = (lax.dot_general(gu_ref[...], w1u_ref[...], dn,
                             preferred_element_type=jnp.float32)
             + lax.dot_general(gi_ref[...], w1i_ref[...], dn,
                               preferred_element_type=jnp.float32)
             + b1_ref[...])
        h = jnp.dot(h, w2_ref[...], preferred_element_type=jnp.float32) \
            + b2_ref[...]
        o = jnp.dot(h, w3_ref[...], preferred_element_type=jnp.float32) \
            + b3_ref[...]
        out_ref[...] = 1.0 / (1.0 + jnp.exp(-o))

    return pl.pallas_call(
        body,
        grid=(B // blk,),
        in_specs=[
            pl.BlockSpec((F, blk), lambda i: (0, i)),
            pl.BlockSpec((F, blk), lambda i: (0, i)),
            pl.BlockSpec((F, n1), lambda i: (0, 0)),
            pl.BlockSpec((F, n1), lambda i: (0, 0)),
            pl.BlockSpec((1, n1), lambda i: (0, 0)),
            pl.BlockSpec((n1, n2), lambda i: (0, 0)),
            pl.BlockSpec((1, n2), lambda i: (0, 0)),
            pl.BlockSpec((n2, 1), lambda i: (0, 0)),
            pl.BlockSpec((1, 1), lambda i: (0, 0)),
        ],
        out_specs=pl.BlockSpec((blk, 1), lambda i: (i, 0)),
        out_shape=jax.ShapeDtypeStruct((B, 1), jnp.float32),
    )(gu, gi, W1u, W1i, b1.reshape(1, n1), W2, b2.reshape(1, n2),
      W3, b3.reshape(1, 1))


def kernel(user, item, user_emb, item_emb, W1, b1, W2, b2, W3, b3):
    B = user.shape[0]
    N = user_emb.shape[0]
    u_flat, i_flat = _flatten(user_emb.T, item_emb.T)
    gather = _make_gather(B, _NP)
    gu, gi = gather(user.astype(jnp.int32), item.astype(jnp.int32),
                    u_flat.reshape(_F, _NP), i_flat.reshape(_F, _NP))
    return _mlp_pallas(gu, gi, W1[:_F], W1[_F:], b1, W2, b2, W3, b3)
